# Initial kernel scaffold; baseline (speedup 1.0000x reference)
import jax
import jax.numpy as jnp
from jax.experimental import pallas as pl


def kernel(x, palette):
    raise NotImplementedError("write your pallas kernel here")


# speedup vs baseline: 48.8857x; 48.8857x over previous
import functools

import jax
import jax.numpy as jnp
from jax import lax
from jax.experimental import pallas as pl
from jax.experimental.pallas import tpu as pltpu
from jax.experimental.pallas import tpu_sc as plsc

_SIZE = 1024
_SCALE = 1024.0
_LOW = 0.0
_MAGIC = 8388608.0
_NC = 2
_NS = 16
_LANES = 16


def _make_sc_call(n_total, hw, chunk):
    nw = _NC * _NS
    per_w = n_total // nw
    n_chunks = per_w // chunk
    mesh = plsc.VectorSubcoreMesh(
        core_axis_name="c", subcore_axis_name="s",
        num_cores=_NC, num_subcores=_NS)

    @functools.partial(
        pl.kernel,
        mesh=mesh,
        compiler_params=pltpu.CompilerParams(needs_layout_passes=False),
        out_type=jax.ShapeDtypeStruct((3 * n_total,), jnp.float32),
        scratch_types=[
            pltpu.VMEM((_SIZE,), jnp.float32),
            pltpu.VMEM((_SIZE,), jnp.float32),
            pltpu.VMEM((_SIZE,), jnp.float32),
            pltpu.VMEM((chunk,), jnp.float32),
            pltpu.VMEM((chunk,), jnp.float32),
            pltpu.VMEM((chunk,), jnp.float32),
            pltpu.VMEM((chunk,), jnp.float32),
        ],
    )
    def sc_colormap(x_hbm, pal_hbm, out_hbm, pal_r, pal_g, pal_b,
                    xv, o_r, o_g, o_b):
        cid = lax.axis_index("c")
        sid = lax.axis_index("s")
        wid = cid * _NS + sid
        pltpu.sync_copy(pal_hbm.at[pl.ds(0, _SIZE)], pal_r)
        pltpu.sync_copy(pal_hbm.at[pl.ds(_SIZE, _SIZE)], pal_g)
        pltpu.sync_copy(pal_hbm.at[pl.ds(2 * _SIZE, _SIZE)], pal_b)

        in_base = wid * per_w
        img = wid * per_w // hw
        rem = in_base - img * hw
        out_base = img * (3 * hw) + rem

        def do_chunk(j, carry):
            off = j * chunk
            pltpu.sync_copy(x_hbm.at[pl.ds(in_base + off, chunk)], xv)

            def inner(i, c2):
                sl = pl.ds(i * _LANES, _LANES)
                t = xv[sl] * _SCALE
                r = (t + _MAGIC) - _MAGIC
                r = jnp.minimum(jnp.maximum(r, 0.0), float(_SIZE - 1))
                idx = r.astype(jnp.int32)
                o_r[sl] = plsc.load_gather(pal_r, [idx])
                o_g[sl] = plsc.load_gather(pal_g, [idx])
                o_b[sl] = plsc.load_gather(pal_b, [idx])
                return c2

            lax.fori_loop(0, chunk // _LANES, inner, 0, unroll=4)
            dst = out_base + off
            pltpu.sync_copy(o_r, out_hbm.at[pl.ds(dst, chunk)])
            pltpu.sync_copy(o_g, out_hbm.at[pl.ds(dst + hw, chunk)])
            pltpu.sync_copy(o_b, out_hbm.at[pl.ds(dst + 2 * hw, chunk)])
            return carry

        lax.fori_loop(0, n_chunks, do_chunk, 0)

    return sc_colormap


def kernel(x, palette):
    b, h, w = x.shape
    hw = h * w
    n_total = b * hw
    chunk = 8192
    call = _make_sc_call(n_total, hw, chunk)
    pal_t = palette.T.reshape(-1).astype(jnp.float32)
    out_flat = call(x.reshape(-1), pal_t)
    return out_flat.reshape(b, 3, h, w)


# speedup vs baseline: 131.4495x; 2.6889x over previous
import functools

import jax
import jax.numpy as jnp
from jax import lax
from jax.experimental import pallas as pl
from jax.experimental.pallas import tpu as pltpu
from jax.experimental.pallas import tpu_sc as plsc

_SIZE = 1024
_SCALE = 1024.0
_MAGIC = 8388608.0
_NC = 2
_NS = 16
_LANES = 16


def _make_sc_call(n_total, hw, chunk):
    nw = _NC * _NS
    per_w = n_total // nw
    n_chunks = per_w // chunk
    mesh = plsc.VectorSubcoreMesh(
        core_axis_name="c", subcore_axis_name="s",
        num_cores=_NC, num_subcores=_NS)

    @functools.partial(
        pl.kernel,
        mesh=mesh,
        compiler_params=pltpu.CompilerParams(needs_layout_passes=False),
        out_type=jax.ShapeDtypeStruct((3 * n_total,), jnp.float32),
        scratch_types=[
            pltpu.VMEM((_SIZE,), jnp.float32),
            pltpu.VMEM((_SIZE,), jnp.float32),
            pltpu.VMEM((_SIZE,), jnp.float32),
            (pltpu.VMEM((chunk,), jnp.float32),) * 2,
            (pltpu.VMEM((chunk,), jnp.float32),) * 2,
            (pltpu.VMEM((chunk,), jnp.float32),) * 2,
            (pltpu.VMEM((chunk,), jnp.float32),) * 2,
            pltpu.SemaphoreType.DMA,
            (pltpu.SemaphoreType.DMA,) * 2,
            (pltpu.SemaphoreType.DMA,) * 2,
        ],
    )
    def sc_colormap(x_hbm, pal_hbm, out_hbm, pal_r, pal_g, pal_b,
                    xring, o_r, o_g, o_b, pal_sem, in_sems, out_sems):
        cid = lax.axis_index("c")
        sid = lax.axis_index("s")
        wid = cid * _NS + sid
        pal_descs = [
            pltpu.async_copy(pal_hbm.at[pl.ds(c * _SIZE, _SIZE)], dst, pal_sem)
            for c, dst in enumerate((pal_r, pal_g, pal_b))]

        in_base = wid * per_w
        img = wid * per_w // hw
        rem = in_base - img * hw
        out_base = img * (3 * hw) + rem

        xbufs = list(xring)
        obufs = [[o_r[p], o_g[p], o_b[p]] for p in (0, 1)]

        in_descs = [None, None]
        out_descs = [None, None]
        in_descs[0] = pltpu.async_copy(
            x_hbm.at[pl.ds(in_base, chunk)], xbufs[0], in_sems[0])
        for d in pal_descs:
            d.wait()

        for j in range(n_chunks):
            p = j & 1
            if j + 1 < n_chunks:
                in_descs[1 - p] = pltpu.async_copy(
                    x_hbm.at[pl.ds(in_base + (j + 1) * chunk, chunk)],
                    xbufs[1 - p], in_sems[1 - p])
            in_descs[p].wait()
            if out_descs[p] is not None:
                for d in out_descs[p]:
                    d.wait()
            xb = xbufs[p]
            ob = obufs[p]

            @plsc.parallel_loop(0, chunk, step=_LANES, unroll=8)
            def _(i):
                sl = pl.ds(i, _LANES)
                t = xb[sl] * _SCALE
                r = (t + _MAGIC) - _MAGIC
                r = jnp.minimum(jnp.maximum(r, 0.0), float(_SIZE - 1))
                idx = r.astype(jnp.int32)
                ob[0][sl] = plsc.load_gather(pal_r, [idx])
                ob[1][sl] = plsc.load_gather(pal_g, [idx])
                ob[2][sl] = plsc.load_gather(pal_b, [idx])

            dst = out_base + j * chunk
            out_descs[p] = [
                pltpu.async_copy(
                    ob[c], out_hbm.at[pl.ds(dst + c * hw, chunk)], out_sems[p])
                for c in range(3)]

        for p in (0, 1):
            if out_descs[p] is not None:
                for d in out_descs[p]:
                    d.wait()

    return sc_colormap


def kernel(x, palette):
    b, h, w = x.shape
    hw = h * w
    n_total = b * hw
    chunk = 8192
    call = _make_sc_call(n_total, hw, chunk)
    pal_t = palette.T.reshape(-1).astype(jnp.float32)
    out_flat = call(x.reshape(-1), pal_t)
    return out_flat.reshape(b, 3, h, w)


# speedup vs baseline: 285.0929x; 2.1688x over previous
import functools

import jax
import jax.numpy as jnp
from jax import lax
from jax.experimental import pallas as pl
from jax.experimental.pallas import tpu as pltpu
from jax.experimental.pallas import tpu_sc as plsc

_SIZE = 1024
_SCALE = 1024.0
_MAGIC = 8388608.0
_NC = 2
_NS = 16
_LANES = 16


def _make_sc_call(batch, h, w, rows_per_chunk, out_shape):
    hw = h * w
    n_total = batch * hw
    nw = _NC * _NS
    per_w = n_total // nw
    chunk = rows_per_chunk * w
    n_chunks = per_w // chunk
    mesh = plsc.VectorSubcoreMesh(
        core_axis_name="c", subcore_axis_name="s",
        num_cores=_NC, num_subcores=_NS)

    @functools.partial(
        pl.kernel,
        mesh=mesh,
        compiler_params=pltpu.CompilerParams(needs_layout_passes=False),
        out_type=jax.ShapeDtypeStruct(out_shape, jnp.float32),
        scratch_types=[
            pltpu.VMEM((_SIZE,), jnp.float32),
            pltpu.VMEM((_SIZE,), jnp.float32),
            pltpu.VMEM((_SIZE,), jnp.float32),
            (pltpu.VMEM((rows_per_chunk, w), jnp.float32),) * 2,
            (pltpu.VMEM((rows_per_chunk, w), jnp.float32),) * 2,
            (pltpu.VMEM((rows_per_chunk, w), jnp.float32),) * 2,
            (pltpu.VMEM((rows_per_chunk, w), jnp.float32),) * 2,
            pltpu.SemaphoreType.DMA,
            (pltpu.SemaphoreType.DMA,) * 2,
            (pltpu.SemaphoreType.DMA,) * 2,
        ],
    )
    def sc_colormap(x_nat, pal_hbm, out_nat, pal_r, pal_g, pal_b,
                    xring, o_r, o_g, o_b, pal_sem, in_sems, out_sems):
        x_hbm = x_nat.reshape(batch * h, w)
        out_hbm = out_nat.reshape(batch * 3 * h, w)
        cid = lax.axis_index("c")
        sid = lax.axis_index("s")
        wid = cid * _NS + sid
        pal_descs = [
            pltpu.async_copy(pal_hbm.at[pl.ds(c * _SIZE, _SIZE)], dst, pal_sem)
            for c, dst in enumerate((pal_r, pal_g, pal_b))]

        in_base = wid * per_w
        img = wid * per_w // hw
        rem_rows = (in_base - img * hw) // w
        in_row = pl.multiple_of(img * h + rem_rows, rows_per_chunk)
        out_row = pl.multiple_of(img * 3 * h + rem_rows, rows_per_chunk)

        xbufs = list(xring)
        obufs = [[o_r[p], o_g[p], o_b[p]] for p in (0, 1)]

        in_descs = [None, None]
        out_descs = [None, None]
        in_descs[0] = pltpu.async_copy(
            x_hbm.at[pl.ds(in_row, rows_per_chunk)], xbufs[0], in_sems[0])
        for d in pal_descs:
            d.wait()

        for j in range(n_chunks):
            p = j & 1
            if j + 1 < n_chunks:
                in_descs[1 - p] = pltpu.async_copy(
                    x_hbm.at[pl.ds(in_row + (j + 1) * rows_per_chunk,
                                   rows_per_chunk)],
                    xbufs[1 - p], in_sems[1 - p])
            in_descs[p].wait()
            if out_descs[p] is not None:
                for d in out_descs[p]:
                    d.wait()
            xb = xbufs[p]
            ob = obufs[p]

            def row_body(r, carry):
                @plsc.parallel_loop(0, w, step=_LANES, unroll=8)
                def _(i):
                    sl = pl.ds(i, _LANES)
                    t = xb[r, sl] * _SCALE
                    rr = (t + _MAGIC) - _MAGIC
                    rr = jnp.minimum(jnp.maximum(rr, 0.0), float(_SIZE - 1))
                    idx = rr.astype(jnp.int32)
                    ob[0][r, sl] = plsc.load_gather(pal_r, [idx])
                    ob[1][r, sl] = plsc.load_gather(pal_g, [idx])
                    ob[2][r, sl] = plsc.load_gather(pal_b, [idx])
                return carry

            lax.fori_loop(0, rows_per_chunk, row_body, 0)

            dst = out_row + j * rows_per_chunk
            out_descs[p] = [
                pltpu.async_copy(
                    ob[c],
                    out_hbm.at[pl.ds(dst + c * h, rows_per_chunk)],
                    out_sems[p])
                for c in range(3)]

        for p in (0, 1):
            if out_descs[p] is not None:
                for d in out_descs[p]:
                    d.wait()

    return sc_colormap


def kernel(x, palette):
    b, h, w = x.shape
    call = _make_sc_call(b, h, w, 16, (b, 3, h, w))
    pal_t = palette.T.reshape(-1).astype(jnp.float32)
    return call(x, pal_t)
